# baseline (speedup 1.0000x reference)
import functools

import jax
import jax.numpy as jnp
from jax import lax
from jax.experimental import pallas as pl
from jax.experimental.pallas import tpu as pltpu
from jax.experimental.pallas import tpu_sc as plsc

NC = 2
NS = 16
NW = NC * NS
CHUNK = 128
NBUF = 8
ROWBLK = 512
L = 16


def _tc_project(pos, vel, W_rel, W_root, W_pred, b_rel, b_pred, npad):
  n, d = pos.shape
  emb = 2 * d
  out_w = W_pred.shape[1]
  grid = (npad + ROWBLK - 1) // ROWBLK

  def body(pos_ref, vel_ref, wrel_ref, wroot_ref, wpred_ref, brel_ref,
           bpred_ref, tab_ref, z_ref):
    wp16 = jnp.concatenate(
        [wpred_ref[...], jnp.zeros((emb, 16 - out_w), jnp.float32)], axis=1)
    c1 = jnp.dot(wrel_ref[...], wp16, preferred_element_type=jnp.float32)
    c2 = jnp.dot(wroot_ref[...], wp16, preferred_element_type=jnp.float32)
    bias = jnp.dot(brel_ref[...], wp16, preferred_element_type=jnp.float32)
    bias = bias + jnp.concatenate(
        [bpred_ref[...], jnp.zeros((1, 16 - out_w), jnp.float32)], axis=1)
    p = pos_ref[...]
    v = vel_ref[...]
    tab_ref[...] = (
        jnp.dot(p, c1[:d], preferred_element_type=jnp.float32)
        + jnp.dot(v, c1[d:], preferred_element_type=jnp.float32))
    z_ref[...] = (
        jnp.dot(p, c2[:d], preferred_element_type=jnp.float32)
        + jnp.dot(v, c2[d:], preferred_element_type=jnp.float32) + bias)

  return pl.pallas_call(
      body,
      grid=(grid,),
      in_specs=[
          pl.BlockSpec((ROWBLK, d), lambda i: (i, 0)),
          pl.BlockSpec((ROWBLK, d), lambda i: (i, 0)),
          pl.BlockSpec((emb, emb), lambda i: (0, 0)),
          pl.BlockSpec((emb, emb), lambda i: (0, 0)),
          pl.BlockSpec((emb, out_w), lambda i: (0, 0)),
          pl.BlockSpec((1, emb), lambda i: (0, 0)),
          pl.BlockSpec((1, out_w), lambda i: (0, 0)),
      ],
      out_specs=[
          pl.BlockSpec((ROWBLK, 16), lambda i: (i, 0)),
          pl.BlockSpec((ROWBLK, 16), lambda i: (i, 0)),
      ],
      out_shape=[
          jax.ShapeDtypeStruct((npad, 16), jnp.float32),
          jax.ShapeDtypeStruct((npad, 16), jnp.float32),
      ],
  )(pos, vel, W_rel, W_root, W_pred, b_rel.reshape(1, emb),
    b_pred.reshape(1, out_w))


def _sc_segment_sum(tab, z16, edge_index, npad, n):
  e = edge_index.shape[1]
  assert e % NW == 0
  ept = e // NW
  nfull = ept // CHUNK
  tail = ept - nfull * CHUNK
  assert tail % L == 0
  cpt = -(-(nfull + (1 if tail else 0)) // NBUF) * NBUF
  rows_pt = npad // NS
  n_dummy = npad - n
  nrounds = cpt // NBUF
  mesh = plsc.VectorSubcoreMesh(core_axis_name="c", subcore_axis_name="s")

  @functools.partial(
      pl.kernel,
      mesh=mesh,
      out_type=jax.ShapeDtypeStruct((NC, npad, 16), jnp.float32),
      compiler_params=pltpu.CompilerParams(use_tc_tiling_on_sc=False),
      scratch_types=[
          pltpu.VMEM((cpt, CHUNK), jnp.int32),
          pltpu.VMEM((cpt, CHUNK), jnp.int32),
          [pltpu.VMEM((CHUNK, 16), jnp.float32)] * NBUF,
          pltpu.VMEM((rows_pt, 16), jnp.float32),
          pltpu.VMEM_SHARED((npad, 16), jnp.float32),
          [pltpu.SemaphoreType.DMA] * NBUF,
          [pltpu.SemaphoreType.DMA] * NBUF,
          pltpu.SemaphoreType.DMA,
      ],
  )
  def sck(tab_hbm, z_hbm, ei_hbm, out_hbm,
          idx_s, idx_d, vals, buf, acc_sh, gsem, ssem, isem):
    c = lax.axis_index("c")
    s = lax.axis_index("s")
    w = c * NS + s
    r0 = s * rows_pt
    rows = pl.ds(r0, rows_pt)
    e0 = w * ept

    def stage(r, carry):
      pltpu.async_copy(ei_hbm.at[0, pl.ds(e0 + r * CHUNK, CHUNK)],
                       idx_s.at[r], isem)
      pltpu.async_copy(ei_hbm.at[1, pl.ds(e0 + r * CHUNK, CHUNK)],
                       idx_d.at[r], isem)
      return carry

    lax.fori_loop(0, nfull, stage, 0)
    if tail:
      pltpu.async_copy(ei_hbm.at[0, pl.ds(e0 + nfull * CHUNK, tail)],
                       idx_s.at[nfull, pl.ds(0, tail)], isem)
      pltpu.async_copy(ei_hbm.at[1, pl.ds(e0 + nfull * CHUNK, tail)],
                       idx_d.at[nfull, pl.ds(0, tail)], isem)

    iota = lax.iota(jnp.int32, L)
    base = iota + CHUNK * s

    def fill(row, col0):
      k = (row * CHUNK + col0) // L
      idx_s[row, pl.ds(col0, L)] = (base + L * k) % n
      idx_d[row, pl.ds(col0, L)] = n + (base + 7 * k) % n_dummy

    if tail:
      for col0 in range(tail, CHUNK, L):
        fill(nfull, col0)
    for row in range(nfull + (1 if tail else 0), cpt):
      for col0 in range(0, CHUNK, L):
        fill(row, col0)

    @pl.when(c == 0)
    def _():
      pltpu.sync_copy(z_hbm.at[rows], buf)

    @pl.when(c != 0)
    def _():
      zv = jnp.zeros((L,), jnp.float32)

      def zbody(r, carry):
        buf[r] = zv
        return carry

      lax.fori_loop(0, rows_pt, zbody, 0)

    pltpu.sync_copy(buf, acc_sh.at[rows])

    def drain(r, carry):
      pltpu.make_async_copy(ei_hbm.at[0, pl.ds(0, CHUNK)], idx_s.at[0],
                            isem).wait()
      pltpu.make_async_copy(ei_hbm.at[0, pl.ds(0, CHUNK)], idx_d.at[0],
                            isem).wait()
      return carry

    lax.fori_loop(0, nfull, drain, 0)
    if tail:
      pltpu.make_async_copy(ei_hbm.at[0, pl.ds(0, tail)],
                            idx_s.at[0, pl.ds(0, tail)], isem).wait()
      pltpu.make_async_copy(ei_hbm.at[0, pl.ds(0, tail)],
                            idx_d.at[0, pl.ds(0, tail)], isem).wait()
    plsc.subcore_barrier()

    for b in range(NBUF):
      pltpu.async_copy(tab_hbm.at[idx_s.at[b]], vals[b], gsem[b])

    def round_body(g, carry):
      scats = []
      for b in range(NBUF):
        pltpu.make_async_copy(tab_hbm.at[pl.ds(0, CHUNK)], vals[b],
                              gsem[b]).wait()
        scats.append(
            pltpu.async_copy(vals[b], acc_sh.at[idx_d.at[g * NBUF + b]],
                             ssem[b], add=True))
      for b in range(NBUF):
        scats[b].wait()
        jn = (g + 1) * NBUF + b

        @pl.when(jn < cpt)
        def _():
          pltpu.async_copy(tab_hbm.at[idx_s.at[jn]], vals[b], gsem[b])

      return carry

    lax.fori_loop(0, nrounds, round_body, 0)
    plsc.subcore_barrier()

    pltpu.sync_copy(acc_sh.at[rows], buf)
    pltpu.sync_copy(buf, out_hbm.at[c, rows])

  return sck(tab, z16, edge_index)


def _tc_combine(partials, n, out_w):
  npad = partials.shape[1]
  grid = (npad + ROWBLK - 1) // ROWBLK

  def body(p_ref, out_ref):
    acc = p_ref[0] + p_ref[1]
    out_ref[...] = acc[:, :out_w]

  return pl.pallas_call(
      body,
      grid=(grid,),
      in_specs=[pl.BlockSpec((NC, ROWBLK, 16), lambda i: (0, i, 0))],
      out_specs=pl.BlockSpec((ROWBLK, out_w), lambda i: (i, 0)),
      out_shape=jax.ShapeDtypeStruct((n, out_w), jnp.float32),
  )(partials)


def kernel(pos, vel, edge_index, W_rel, b_rel, W_root, W_pred, b_pred):
  n, d = pos.shape
  out_w = W_pred.shape[1]

  rows_pt = -(-(n + 64) // (NS * 8)) * 8
  npad = NS * rows_pt

  tab, z16 = _tc_project(pos, vel, W_rel, W_root, W_pred, b_rel, b_pred, npad)
  partials = _sc_segment_sum(tab, z16, edge_index, npad, n)
  return _tc_combine(partials, n, out_w)
